# baseline (speedup 1.0000x reference)
import functools

import jax
import jax.numpy as jnp
from jax import lax
from jax.experimental import pallas as pl
from jax.experimental.pallas import tpu as pltpu
from jax.experimental.pallas import tpu_sc as plsc

_N = 6400000
_NW = 32
_PER_W = _N // _NW
_CHUNK = 25000
_NCH = _PER_W // _CHUNK

_mesh = plsc.VectorSubcoreMesh(core_axis_name="c", subcore_axis_name="s")


@functools.partial(
    pl.kernel,
    mesh=_mesh,
    out_type=jax.ShapeDtypeStruct((_N,), jnp.float32),
    scratch_types=[
        pltpu.VMEM((_CHUNK,), jnp.float32),
    ],
)
def _sc_copy(in_hbm, out_hbm, buf):
    wid = lax.axis_index("s") * 2 + lax.axis_index("c")
    base = wid * _PER_W
    for t in range(_NCH):
        off = base + t * _CHUNK
        pltpu.sync_copy(in_hbm.at[pl.ds(off, _CHUNK)], buf)
        pltpu.sync_copy(buf, out_hbm.at[pl.ds(off, _CHUNK)])


def kernel(Zj):
    return _sc_copy(Zj)


# speedup vs baseline: 1.0989x; 1.0989x over previous
import functools

import jax
import jax.numpy as jnp
from jax import lax
from jax.experimental import pallas as pl
from jax.experimental.pallas import tpu as pltpu
from jax.experimental.pallas import tpu_sc as plsc

_N = 6400000
_NW = 32
_PER_W = _N // _NW
_CHUNK = 50000
_NCH = _PER_W // _CHUNK

_mesh = plsc.VectorSubcoreMesh(core_axis_name="c", subcore_axis_name="s")


@functools.partial(
    pl.kernel,
    mesh=_mesh,
    out_type=jax.ShapeDtypeStruct((_N,), jnp.float32),
    scratch_types=[
        pltpu.VMEM((_CHUNK,), jnp.float32),
        pltpu.VMEM((_CHUNK,), jnp.float32),
        pltpu.SemaphoreType.DMA((2,)),
        pltpu.SemaphoreType.DMA((2,)),
    ],
)
def _sc_copy(in_hbm, out_hbm, buf0, buf1, isems, osems):
    wid = lax.axis_index("s") * 2 + lax.axis_index("c")
    base = wid * _PER_W
    bufs = (buf0, buf1)

    def in_copy(t, b):
        return pltpu.make_async_copy(
            in_hbm.at[pl.ds(base + t * _CHUNK, _CHUNK)], bufs[b],
            isems.at[b])

    def out_copy(t, b):
        return pltpu.make_async_copy(
            bufs[b], out_hbm.at[pl.ds(base + t * _CHUNK, _CHUNK)],
            osems.at[b])

    in_copy(0, 0).start()
    for t in range(_NCH):
        b = t % 2
        in_copy(t, b).wait()
        out_copy(t, b).start()
        if t + 1 < _NCH:
            if t >= 1:
                out_copy(t - 1, 1 - b).wait()
            in_copy(t + 1, 1 - b).start()
    out_copy(_NCH - 2, _NCH % 2).wait()
    out_copy(_NCH - 1, 1 - _NCH % 2).wait()


def kernel(Zj):
    return _sc_copy(Zj)


# speedup vs baseline: 2.6697x; 2.4294x over previous
import jax
import jax.numpy as jnp
from jax.experimental import pallas as pl
from jax.experimental.pallas import tpu as pltpu

_N = 6400000
_LANES = 128
_ROWS = _N // _LANES
_NCHUNK = 2
_CROWS = _ROWS // _NCHUNK


def _copy_body(in_ref, out_ref, buf0, buf1, isems, osems):
    bufs = (buf0, buf1)

    def in_copy(i):
        return pltpu.make_async_copy(
            in_ref.at[pl.ds(i * _CROWS, _CROWS)], bufs[i], isems.at[i])

    def out_copy(i):
        return pltpu.make_async_copy(
            bufs[i], out_ref.at[pl.ds(i * _CROWS, _CROWS)], osems.at[i])

    in_copy(0).start()
    in_copy(1).start()
    in_copy(0).wait()
    out_copy(0).start()
    in_copy(1).wait()
    out_copy(1).start()
    out_copy(0).wait()
    out_copy(1).wait()


def kernel(Zj):
    x = Zj.reshape(_ROWS, _LANES)
    y = pl.pallas_call(
        _copy_body,
        out_shape=jax.ShapeDtypeStruct((_ROWS, _LANES), Zj.dtype),
        in_specs=[pl.BlockSpec(memory_space=pl.ANY)],
        out_specs=pl.BlockSpec(memory_space=pl.ANY),
        scratch_shapes=[
            pltpu.VMEM((_CROWS, _LANES), jnp.float32),
            pltpu.VMEM((_CROWS, _LANES), jnp.float32),
            pltpu.SemaphoreType.DMA((2,)),
            pltpu.SemaphoreType.DMA((2,)),
        ],
    )(x)
    return y.reshape(_N)
